# baseline (speedup 1.0000x reference)
import functools

import jax
import jax.numpy as jnp
from jax import lax
from jax.experimental import pallas as pl
from jax.experimental.pallas import tpu as pltpu
from jax.experimental.pallas import tpu_sc as plsc

V = 100000
D = 128
C = 20
CPAD = 24
N = 128
VB = 12800
NBLK = (V + VB - 1) // VB
PADV = NBLK * VB

_sc_mesh = plsc.VectorSubcoreMesh(core_axis_name="c", subcore_axis_name="s")


@functools.partial(
    pl.kernel,
    out_type=jax.ShapeDtypeStruct((CPAD, D), jnp.float32),
    mesh=_sc_mesh,
    scratch_types=[
        pltpu.VMEM((CPAD,), jnp.int32),
        pltpu.VMEM((CPAD, D), jnp.float32),
        pltpu.SemaphoreType.DMA,
    ],
)
def _sc_gather(idx_hbm, table_hbm, out_hbm, idx_v, rows_v, sem):
    wid = lax.axis_index("s") * 2 + lax.axis_index("c")

    @pl.when(wid == 0)
    def _():
        pltpu.sync_copy(idx_hbm, idx_v)
        pltpu.async_copy(table_hbm.at[idx_v], rows_v, sem).wait()
        pltpu.sync_copy(rows_v, out_hbm)


def _fused_kernel(g_ref, w1_ref, b1_ref, w2_ref, b2_ref, out_ref, h_ref):
    i = pl.program_id(0)

    @pl.when(i == 0)
    def _hidden():
        acc = b1_ref[...].astype(jnp.float32)
        for p in range(C):
            acc = acc + lax.dot_general(
                g_ref[pl.ds(p, 1), :],
                w1_ref[:, pl.ds(p * D, D)],
                (((1,), (1,)), ((), ())),
                preferred_element_type=jnp.float32,
            )
        h_ref[...] = jnp.maximum(acc, 0.0)

    lb = lax.dot_general(
        h_ref[...],
        w2_ref[...],
        (((1,), (1,)), ((), ())),
        preferred_element_type=jnp.float32,
    ) + b2_ref[...]

    @pl.when(i < NBLK - 1)
    def _store():
        out_ref[0:1, pl.ds(i * VB, VB)] = lb

    @pl.when(i == NBLK - 1)
    def _store_masked_and_normalize():
        cols = (NBLK - 1) * VB + lax.broadcasted_iota(jnp.int32, (1, VB), 1)
        out_ref[0:1, pl.ds((NBLK - 1) * VB, VB)] = jnp.where(cols < V, lb, -1e30)
        scr = out_ref[...]
        m = jnp.max(scr, axis=1, keepdims=True)
        s = jnp.sum(jnp.exp(scr - m), axis=1, keepdims=True)
        out_ref[...] = scr - (m + jnp.log(s))


def kernel(inputs, emb, W1, b1, W2, b2):
    b1r = b1.reshape(1, N)
    b2r = b2.reshape(1, V)
    idx = jnp.concatenate([inputs, jnp.zeros((CPAD - C,), jnp.int32)])

    g = _sc_gather(idx, emb)

    out = pl.pallas_call(
        _fused_kernel,
        grid=(NBLK,),
        in_specs=[
            pl.BlockSpec((CPAD, D), lambda i: (0, 0)),
            pl.BlockSpec((N, C * D), lambda i: (0, 0)),
            pl.BlockSpec((1, N), lambda i: (0, 0)),
            pl.BlockSpec((VB, D), lambda i: (i, 0)),
            pl.BlockSpec((1, VB), lambda i: (0, i)),
        ],
        out_specs=pl.BlockSpec((1, PADV), lambda i: (0, 0)),
        out_shape=jax.ShapeDtypeStruct((1, PADV), jnp.float32),
        scratch_shapes=[
            pltpu.VMEM((1, N), jnp.float32),
        ],
        compiler_params=pltpu.CompilerParams(
            dimension_semantics=("arbitrary",),
            vmem_limit_bytes=100 * 1024 * 1024,
        ),
    )(g, W1, b1r, W2, b2r)
    return out[:, :V]


# speedup vs baseline: 1.6706x; 1.6706x over previous
import jax
import jax.numpy as jnp
from jax import lax
from jax.experimental import pallas as pl
from jax.experimental.pallas import tpu as pltpu

V = 100000
D = 128
C = 20
N = 128
CH = 3200
NCH = (V + CH - 1) // CH
PADV = NCH * CH
K = 4
LAST = V - (NCH - 1) * CH


def _w2_copy(w2_hbm, bufs, sems, c):
    rows = LAST if c == NCH - 1 else CH
    q = c % K
    return pltpu.make_async_copy(
        w2_hbm.at[pl.ds(c * CH, rows), :],
        bufs.at[q, pl.ds(0, rows), :],
        sems.at[q],
    )


def _fused_kernel(idx_ref, emb_ref, w1_ref, b1_ref, w2_ref, b2_ref,
                  out_ref, g_ref, bufs_ref, gsem, sems):
    for p in range(C):
        pltpu.make_async_copy(
            emb_ref.at[pl.ds(idx_ref[p], 1), :],
            g_ref.at[pl.ds(p, 1), :],
            gsem,
        ).start()
    for c in range(K):
        _w2_copy(w2_ref, bufs_ref, sems, c).start()

    for p in range(C):
        pltpu.make_async_copy(
            emb_ref.at[pl.ds(idx_ref[p], 1), :],
            g_ref.at[pl.ds(p, 1), :],
            gsem,
        ).wait()
    acc = b1_ref[...].astype(jnp.float32)
    for p in range(C):
        acc = acc + lax.dot_general(
            g_ref[pl.ds(p, 1), :],
            w1_ref[:, pl.ds(p * D, D)],
            (((1,), (1,)), ((), ())),
            preferred_element_type=jnp.float32,
        )
    h = jnp.maximum(acc, 0.0)

    for c in range(NCH):
        _w2_copy(w2_ref, bufs_ref, sems, c).wait()
        lb = lax.dot_general(
            h,
            bufs_ref[c % K],
            (((1,), (1,)), ((), ())),
            preferred_element_type=jnp.float32,
        ) + b2_ref[0:1, pl.ds(c * CH, CH)]
        if c == NCH - 1:
            cols = c * CH + lax.broadcasted_iota(jnp.int32, (1, CH), 1)
            lb = jnp.where(cols < V, lb, -1e30)
        out_ref[0:1, pl.ds(c * CH, CH)] = lb
        if c + K < NCH:
            _w2_copy(w2_ref, bufs_ref, sems, c + K).start()

    scr = out_ref[...]
    m = jnp.max(scr, axis=1, keepdims=True)
    s = jnp.sum(jnp.exp(scr - m), axis=1, keepdims=True)
    out_ref[...] = scr - (m + jnp.log(s))


def kernel(inputs, emb, W1, b1, W2, b2):
    b1r = b1.reshape(1, N)
    b2r = jnp.pad(b2.reshape(1, V), ((0, 0), (0, PADV - V)),
                  constant_values=-1e30)
    out = pl.pallas_call(
        _fused_kernel,
        in_specs=[
            pl.BlockSpec(memory_space=pltpu.MemorySpace.SMEM),
            pl.BlockSpec(memory_space=pltpu.MemorySpace.HBM),
            pl.BlockSpec((N, C * D), lambda: (0, 0)),
            pl.BlockSpec((1, N), lambda: (0, 0)),
            pl.BlockSpec(memory_space=pltpu.MemorySpace.HBM),
            pl.BlockSpec((1, PADV), lambda: (0, 0)),
        ],
        out_specs=pl.BlockSpec((1, PADV), lambda: (0, 0)),
        out_shape=jax.ShapeDtypeStruct((1, PADV), jnp.float32),
        scratch_shapes=[
            pltpu.VMEM((C, D), jnp.float32),
            pltpu.VMEM((K, CH, D), jnp.float32),
            pltpu.SemaphoreType.DMA,
            pltpu.SemaphoreType.DMA((K,)),
        ],
        compiler_params=pltpu.CompilerParams(
            vmem_limit_bytes=100 * 1024 * 1024,
        ),
    )(inputs, emb, W1, b1r, W2, b2r)
    return out[:, :V]
